# Initial kernel scaffold; baseline (speedup 1.0000x reference)
import jax
import jax.numpy as jnp
from jax.experimental import pallas as pl


def kernel(data, W1, b1, W2, b2):
    raise NotImplementedError("write your pallas kernel here")


# speedup vs baseline: 2.9092x; 2.9092x over previous
import functools

import jax
import jax.numpy as jnp
from jax.experimental import pallas as pl

PHI = 3.1415926 * 0.3
MIN_NORM = 1e-15
PROJ_EPS = 4e-3
A_FMRI = 0.5

B, T, N, H = 16, 150, 200, 128


def _fkernel(x):
    norm = jnp.sqrt(jnp.sum(x * x, axis=-1, keepdims=True))
    norm = jnp.maximum(norm, MIN_NORM)
    maxnorm = 1.0 - PROJ_EPS
    p = jnp.where(norm > maxnorm, x * (maxnorm / norm), x)
    p_norm = jnp.minimum(norm, maxnorm)
    z = jnp.clip(p_norm, -1.0 + 1e-7, 1.0 - 1e-7)
    scale = 0.5 * jnp.log((1.0 + z) / (1.0 - z)) / p_norm
    return scale * p


def _act(x):
    return jnp.maximum(x, 0.0) + A_FMRI * jnp.cos(x + PHI)


def _gcn_kernel(data_ref, w1_ref, b1_ref, w2_ref, b2_ref, x_ref, adj_ref):
    xb = data_ref[0]
    xc = xb - jnp.mean(xb, axis=0, keepdims=True)
    c = jax.lax.dot_general(
        xc, xc, (((0,), (0,)), ((), ())), preferred_element_type=jnp.float32
    )
    rows = jax.lax.broadcasted_iota(jnp.int32, (N, N), 0)
    cols = jax.lax.broadcasted_iota(jnp.int32, (N, N), 1)
    eye = rows == cols
    d2 = jnp.sum(jnp.where(eye, c, 0.0), axis=1, keepdims=True)
    d = jnp.sqrt(jnp.maximum(d2, 0.0))
    denom = d * d.reshape(1, N)
    corr = jnp.where(denom > 0.0, c / denom, 0.0)
    adj = jnp.abs(jnp.clip(corr, -1.0, 1.0))
    adj_ref[0] = adj

    deg = jnp.sum(adj, axis=1, keepdims=True) + 1.0
    dinv = jax.lax.rsqrt(deg)
    a_nor = (adj + jnp.where(eye, 1.0, 0.0)) * dinv * dinv.reshape(1, N)

    w1 = w1_ref[...]
    w2 = w2_ref[...]

    h1 = jnp.dot(adj, w1, preferred_element_type=jnp.float32)
    x1 = jnp.dot(a_nor, h1, preferred_element_type=jnp.float32) + b1_ref[...]
    g1 = _act(_fkernel(x1))

    h2 = jnp.dot(g1, w2, preferred_element_type=jnp.float32)
    x2 = jnp.dot(a_nor, h2, preferred_element_type=jnp.float32) + b2_ref[...]
    x_ref[0] = _act(_fkernel(x2))


@functools.partial(jax.jit, static_argnames=())
def kernel(data, W1, b1, W2, b2):
    b1r = b1.reshape(1, H)
    b2r = b2.reshape(1, H)
    x, adj = pl.pallas_call(
        _gcn_kernel,
        grid=(B,),
        in_specs=[
            pl.BlockSpec((1, T, N), lambda b: (b, 0, 0)),
            pl.BlockSpec((N, H), lambda b: (0, 0)),
            pl.BlockSpec((1, H), lambda b: (0, 0)),
            pl.BlockSpec((H, H), lambda b: (0, 0)),
            pl.BlockSpec((1, H), lambda b: (0, 0)),
        ],
        out_specs=[
            pl.BlockSpec((1, N, H), lambda b: (b, 0, 0)),
            pl.BlockSpec((1, N, N), lambda b: (b, 0, 0)),
        ],
        out_shape=[
            jax.ShapeDtypeStruct((B, N, H), jnp.float32),
            jax.ShapeDtypeStruct((B, N, N), jnp.float32),
        ],
    )(data, W1, b1r, W2, b2r)
    return (x, adj)
